# baseline (speedup 1.0000x reference)
import jax
import jax.numpy as jnp
from jax.experimental import pallas as pl

N = 10000
E = 320000
FDIM = 128
K = 64
SR_CUT = 10.0
NB = 3
NRI = 2
NRA = 2
NRO = 1


def _ssp(x):
    return jax.nn.softplus(x) - 0.6931471805599453


def _sqrt_kernel(d2_ref, out_ref):
    out_ref[...] = jnp.sqrt(jax.nn.relu(d2_ref[...]))


def kernel(Z, R, idx_i, idx_j, params):
    Ri = R[idx_i]
    Rj = R[idx_j]
    d2 = jnp.sum((Ri - Rj) ** 2, -1)
    Dij = pl.pallas_call(
        _sqrt_kernel,
        out_shape=jax.ShapeDtypeStruct((E,), jnp.float32),
        grid=(1,),
        in_specs=[pl.BlockSpec((E,), lambda i: (0,))],
        out_specs=pl.BlockSpec((E,), lambda i: (0,)),
    )(d2)
    xr = Dij / SR_CUT
    cut = jnp.where(Dij < SR_CUT, 1.0 - 6.0 * xr ** 5 + 15.0 * xr ** 4 - 10.0 * xr ** 3, 0.0)
    mu = jax.nn.softplus(params['centers'])
    beta = jax.nn.softplus(params['widths'])
    rbf = cut[:, None] * jnp.exp(-beta[None, :] * (jnp.exp(-Dij)[:, None] - mu[None, :]) ** 2)
    x = params['emb'][Z]
    Ea = jnp.zeros((N,), jnp.float32)
    Qa = jnp.zeros((N,), jnp.float32)
    nhloss = jnp.zeros((), jnp.float32)
    lastout2 = None
    for b in range(NB):
        g = rbf @ params['k2f'][b]
        xa = _ssp(x)
        xi = xa @ params['Wi'][b] + params['bi'][b]
        xj = g * (xa @ params['Wj'][b] + params['bj'][b])[idx_j]
        m = xi + jax.ops.segment_sum(xj, idx_i, num_segments=N)
        for r in range(NRI):
            y = _ssp(_ssp(m) @ params['riW1'][b, r] + params['rib1'][b, r])
            m = m + y @ params['riW2'][b, r] + params['rib2'][b, r]
        m = _ssp(m)
        x = params['u'][b] * x + m @ params['projW'][b] + params['projb'][b]
        for r in range(NRA):
            y = _ssp(_ssp(x) @ params['raW1'][b, r] + params['rab1'][b, r])
            x = x + y @ params['raW2'][b, r] + params['rab2'][b, r]
        o = x
        for r in range(NRO):
            y = _ssp(_ssp(o) @ params['roW1'][b, r] + params['rob1'][b, r])
            o = o + y @ params['roW2'][b, r] + params['rob2'][b, r]
        out = _ssp(o) @ params['outW'][b] + params['outb'][b]
        Ea = Ea + out[:, 0]
        Qa = Qa + out[:, 1]
        out2 = out ** 2
        if b > 0:
            nhloss = nhloss + jnp.mean(out2 / (out2 + lastout2 + 1e-07))
        lastout2 = out2
    Ea = params['Escale'][Z] * Ea + params['Eshift'][Z]
    Qa = params['Qscale'][Z] * Qa + params['Qshift'][Z]
    return (Ea, Qa, Dij, nhloss)


# speedup vs baseline: 3.3194x; 3.3194x over previous
import functools

import jax
import jax.numpy as jnp
from jax import lax
from jax.experimental import pallas as pl
from jax.experimental.pallas import tpu as pltpu
from jax.experimental.pallas import tpu_sc as plsc

N = 10000
E = 320000
FDIM = 128
K = 64
SR_CUT = 10.0
NB = 3
NRI = 2
NRA = 2
NRO = 1
LN2 = 0.6931471805599453

NC, NS, L = 2, 16, 16
NW = NC * NS
CE = 128
NCH = E // CE
ROWS_Q, ROWS_R = divmod(NCH, NW)
EPW_MAX = (ROWS_Q + 1) * CE
NP = 10240
NPS = NP // NS
ZCH = 128
BN = 2000
GE = 2000


def _ssp(v):
    return jnp.maximum(v, 0.0) + jnp.log1p(jnp.exp(-jnp.abs(v))) - LN2


def _softplus(v):
    return jnp.maximum(v, 0.0) + jnp.log1p(jnp.exp(-jnp.abs(v)))


def _d2_body(rx_hbm, ry_hbm, rz_hbm, ii_hbm, ij_hbm, d2_hbm,
             rx, ry, rz, ii, ij, d2):
    wid = lax.axis_index("s") * NC + lax.axis_index("c")
    base = (wid * ROWS_Q + jnp.minimum(wid, ROWS_R)) * CE
    extra = wid < ROWS_R
    nmain = ROWS_Q * CE
    pltpu.sync_copy(rx_hbm, rx)
    pltpu.sync_copy(ry_hbm, ry)
    pltpu.sync_copy(rz_hbm, rz)
    pltpu.sync_copy(ii_hbm.at[pl.ds(base, nmain)], ii.at[pl.ds(0, nmain)])
    pltpu.sync_copy(ij_hbm.at[pl.ds(base, nmain)], ij.at[pl.ds(0, nmain)])

    @pl.when(extra)
    def _():
        pltpu.sync_copy(ii_hbm.at[pl.ds(base + nmain, CE)],
                        ii.at[pl.ds(nmain, CE)])
        pltpu.sync_copy(ij_hbm.at[pl.ds(base + nmain, CE)],
                        ij.at[pl.ds(nmain, CE)])

    def body(k, _):
        sl = pl.ds(k * L, L)
        a = ii[sl]
        b = ij[sl]
        dx = plsc.load_gather(rx, [a]) - plsc.load_gather(rx, [b])
        dy = plsc.load_gather(ry, [a]) - plsc.load_gather(ry, [b])
        dz = plsc.load_gather(rz, [a]) - plsc.load_gather(rz, [b])
        d2[sl] = dx * dx + dy * dy + dz * dz
        return 0

    nedge = nmain + jnp.where(extra, CE, 0)
    lax.fori_loop(0, nedge // L, body, 0)
    pltpu.sync_copy(d2.at[pl.ds(0, nmain)], d2_hbm.at[pl.ds(base, nmain)])

    @pl.when(extra)
    def _():
        pltpu.sync_copy(d2.at[pl.ds(nmain, CE)],
                        d2_hbm.at[pl.ds(base + nmain, CE)])


def _make_d2():
    mesh = plsc.VectorSubcoreMesh(core_axis_name="c", subcore_axis_name="s",
                                  num_cores=NC, num_subcores=NS)
    return pl.kernel(
        _d2_body,
        out_type=jax.ShapeDtypeStruct((E,), jnp.float32),
        mesh=mesh,
        scratch_types=[
            pltpu.VMEM((N,), jnp.float32),
            pltpu.VMEM((N,), jnp.float32),
            pltpu.VMEM((N,), jnp.float32),
            pltpu.VMEM((EPW_MAX,), jnp.int32),
            pltpu.VMEM((EPW_MAX,), jnp.int32),
            pltpu.VMEM((EPW_MAX,), jnp.float32),
        ],
        compiler_params=pltpu.CompilerParams(needs_layout_passes=False),
    )


def _seg_body(g3, hj_hbm, ii_hbm, ij_hbm, macc_hbm,
              idxi, idxj, hjg, gbuf, acc, sem):
    cid = lax.axis_index("c")
    sid = lax.axis_index("s")
    wid = sid * NC + cid

    def zrow(r, _):
        for c in range(FDIM // L):
            hjg[r, pl.ds(c * L, L)] = jnp.zeros((L,), jnp.float32)
        return 0

    lax.fori_loop(0, ZCH, zrow, 0)
    for jj in range(NPS // ZCH):
        pltpu.sync_copy(hjg, acc.at[pl.ds(sid * NPS + jj * ZCH, ZCH)])
    plsc.subcore_barrier()

    base = wid * ROWS_Q + jnp.minimum(wid, ROWS_R)
    count = ROWS_Q + jnp.where(wid < ROWS_R, 1, 0)

    def chunk(k, _):
        r = base + k
        pltpu.sync_copy(ij_hbm.at[pl.ds(r * CE, CE)], idxj)
        pltpu.async_copy(hj_hbm.at[idxj], hjg, sem).wait()
        pltpu.sync_copy(g3.at[r], gbuf)

        def mrow(rr, _):
            for c in range(FDIM // L):
                sl = pl.ds(c * L, L)
                hjg[rr, sl] = hjg[rr, sl] * gbuf[rr, sl]
            return 0

        lax.fori_loop(0, CE, mrow, 0)
        pltpu.sync_copy(ii_hbm.at[pl.ds(r * CE, CE)], idxi)
        pltpu.sync_copy(hjg, acc.at[idxi], add=True)
        return 0

    lax.fori_loop(0, count, chunk, 0)
    plsc.subcore_barrier()
    for jj in range(NPS // ZCH):
        r0 = sid * NPS + jj * ZCH
        pltpu.sync_copy(acc.at[pl.ds(r0, ZCH)],
                        macc_hbm.at[cid, pl.ds(r0, ZCH)])


def _make_seg():
    mesh = plsc.VectorSubcoreMesh(core_axis_name="c", subcore_axis_name="s",
                                  num_cores=NC, num_subcores=NS)
    return pl.kernel(
        _seg_body,
        out_type=jax.ShapeDtypeStruct((NC, NP, FDIM), jnp.float32),
        mesh=mesh,
        scratch_types=[
            pltpu.VMEM((CE,), jnp.int32),
            pltpu.VMEM((CE,), jnp.int32),
            pltpu.VMEM((CE, FDIM), jnp.float32),
            pltpu.VMEM((CE, FDIM), jnp.float32),
            pltpu.VMEM_SHARED((NP, FDIM), jnp.float32),
            pltpu.SemaphoreType.DMA,
        ],
        compiler_params=pltpu.CompilerParams(needs_layout_passes=False),
    )


def _g_body(d2_ref, cen_ref, wid_ref, k2f_ref, g_ref, dij_ref):
    d2 = d2_ref[...]
    dij = jnp.sqrt(jnp.maximum(d2, 0.0))
    dij_ref[...] = dij
    xr = dij * (1.0 / SR_CUT)
    xr2 = xr * xr
    xr3 = xr2 * xr
    cut = 1.0 + ((15.0 - 6.0 * xr) * xr - 10.0) * xr3
    cut = jnp.where(dij < SR_CUT, cut, 0.0)
    mu = _softplus(cen_ref[...])
    beta = _softplus(wid_ref[...])
    diff = jnp.exp(-dij) - mu
    rbf = cut * jnp.exp(-beta * diff * diff)
    g_ref[...] = jnp.dot(rbf, k2f_ref[...],
                         preferred_element_type=jnp.float32)


def _run_g(d2c, centers_r, widths_r, k2f_b):
    return pl.pallas_call(
        _g_body,
        grid=(E // GE,),
        in_specs=[
            pl.BlockSpec((GE, 1), lambda i: (i, 0)),
            pl.BlockSpec((1, K), lambda i: (0, 0)),
            pl.BlockSpec((1, K), lambda i: (0, 0)),
            pl.BlockSpec((K, FDIM), lambda i: (0, 0)),
        ],
        out_specs=[
            pl.BlockSpec((GE, FDIM), lambda i: (i, 0)),
            pl.BlockSpec((GE, 1), lambda i: (i, 0)),
        ],
        out_shape=[
            jax.ShapeDtypeStruct((E, FDIM), jnp.float32),
            jax.ShapeDtypeStruct((E, 1), jnp.float32),
        ],
    )(d2c, centers_r, widths_r, k2f_b)


def _c0_body(z_ref, emb_ref, wi_ref, bi_ref, wj_ref, bj_ref,
             x_ref, xi_ref, hj_ref):
    zb = z_ref[...]
    iota = lax.broadcasted_iota(jnp.int32, (BN, FDIM), 1)
    oh = (iota == zb).astype(jnp.float32)
    x = jnp.dot(oh, emb_ref[...], preferred_element_type=jnp.float32)
    x_ref[...] = x
    xa = _ssp(x)
    xi_ref[...] = jnp.dot(xa, wi_ref[...],
                          preferred_element_type=jnp.float32) + bi_ref[...]
    hj_ref[...] = jnp.dot(xa, wj_ref[...],
                          preferred_element_type=jnp.float32) + bj_ref[...]


def _run_c0(z2, emb_pad, wi, bi, wj, bj):
    full = lambda shape: pl.BlockSpec(shape, lambda i: (0,) * len(shape))
    return pl.pallas_call(
        _c0_body,
        grid=(N // BN,),
        in_specs=[
            pl.BlockSpec((BN, 1), lambda i: (i, 0)),
            full((FDIM, FDIM)),
            full((FDIM, FDIM)),
            full((1, FDIM)),
            full((FDIM, FDIM)),
            full((1, FDIM)),
        ],
        out_specs=[pl.BlockSpec((BN, FDIM), lambda i: (i, 0))] * 3,
        out_shape=[jax.ShapeDtypeStruct((N, FDIM), jnp.float32)] * 3,
    )(z2, emb_pad, wi, bi, wj, bj)


def _node_body(has_next, xi_ref, macc_ref, x_ref,
               riW1_ref, rib1_ref, riW2_ref, rib2_ref,
               projW_ref, projb_ref, u_ref,
               raW1_ref, rab1_ref, raW2_ref, rab2_ref,
               roW1_ref, rob1_ref, roW2_ref, rob2_ref,
               outW_ref, outb_ref, wiN_ref, biN_ref, wjN_ref, bjN_ref,
               *out_refs):
    dot = functools.partial(jnp.dot, preferred_element_type=jnp.float32)
    m = xi_ref[...] + macc_ref[0] + macc_ref[1]
    for r in range(NRI):
        y = _ssp(dot(_ssp(m), riW1_ref[r]) + rib1_ref[r])
        m = m + dot(y, riW2_ref[r]) + rib2_ref[r]
    m = _ssp(m)
    x = u_ref[...] * x_ref[...] + dot(m, projW_ref[...]) + projb_ref[...]
    for r in range(NRA):
        y = _ssp(dot(_ssp(x), raW1_ref[r]) + rab1_ref[r])
        x = x + dot(y, raW2_ref[r]) + rab2_ref[r]
    o = x
    for r in range(NRO):
        y = _ssp(dot(_ssp(o), roW1_ref[r]) + rob1_ref[r])
        o = o + dot(y, roW2_ref[r]) + rob2_ref[r]
    out_refs[0][...] = dot(_ssp(o), outW_ref[...]) + outb_ref[...]
    if has_next:
        out_refs[1][...] = x
        xa = _ssp(x)
        out_refs[2][...] = dot(xa, wiN_ref[...],
                               preferred_element_type=jnp.float32) + biN_ref[...]
        out_refs[3][...] = dot(xa, wjN_ref[...],
                               preferred_element_type=jnp.float32) + bjN_ref[...]


def _run_node(has_next, xi, macc, x, wts):
    full = lambda shape: pl.BlockSpec(shape, lambda i: (0,) * len(shape))
    wspecs = [
        full((NRI, FDIM, FDIM)), full((NRI, 1, FDIM)),
        full((NRI, FDIM, FDIM)), full((NRI, 1, FDIM)),
        full((FDIM, FDIM)), full((1, FDIM)), full((1, FDIM)),
        full((NRA, FDIM, FDIM)), full((NRA, 1, FDIM)),
        full((NRA, FDIM, FDIM)), full((NRA, 1, FDIM)),
        full((NRO, FDIM, FDIM)), full((NRO, 1, FDIM)),
        full((NRO, FDIM, FDIM)), full((NRO, 1, FDIM)),
        full((FDIM, 2)), full((1, 2)),
        full((FDIM, FDIM)), full((1, FDIM)),
        full((FDIM, FDIM)), full((1, FDIM)),
    ]
    n_out = 4 if has_next else 1
    out_specs = [pl.BlockSpec((BN, 2), lambda i: (i, 0))] + \
        [pl.BlockSpec((BN, FDIM), lambda i: (i, 0))] * (n_out - 1)
    out_shape = [jax.ShapeDtypeStruct((N, 2), jnp.float32)] + \
        [jax.ShapeDtypeStruct((N, FDIM), jnp.float32)] * (n_out - 1)
    return pl.pallas_call(
        functools.partial(_node_body, has_next),
        grid=(N // BN,),
        in_specs=[
            pl.BlockSpec((BN, FDIM), lambda i: (i, 0)),
            pl.BlockSpec((NC, BN, FDIM), lambda i: (0, i, 0)),
            pl.BlockSpec((BN, FDIM), lambda i: (i, 0)),
        ] + wspecs,
        out_specs=out_specs,
        out_shape=out_shape,
    )(xi, macc, x, *wts)


def _fin_body(z_ref, o0_ref, o1_ref, o2_ref,
              esc_ref, esh_ref, qsc_ref, qsh_ref,
              ea_ref, qa_ref, nh_ref):
    i = pl.program_id(0)
    zb = z_ref[...]
    iota = lax.broadcasted_iota(jnp.int32, (BN, FDIM), 1)
    oh = (iota == zb).astype(jnp.float32)
    dot = functools.partial(jnp.dot, preferred_element_type=jnp.float32)
    o0 = o0_ref[...]
    o1 = o1_ref[...]
    o2 = o2_ref[...]
    s = o0 + o1 + o2
    ea_ref[...] = dot(oh, esc_ref[...]) * s[:, 0:1] + dot(oh, esh_ref[...])
    qa_ref[...] = dot(oh, qsc_ref[...]) * s[:, 1:2] + dot(oh, qsh_ref[...])
    p0 = o0 * o0
    p1 = o1 * o1
    p2 = o2 * o2
    part = jnp.sum(p1 / (p1 + p0 + 1e-07)) + jnp.sum(p2 / (p2 + p1 + 1e-07))

    @pl.when(i == 0)
    def _():
        nh_ref[...] = jnp.zeros((1, 1), jnp.float32)

    nh_ref[...] += part * (1.0 / (2.0 * N))


def _run_fin(z2, o0, o1, o2, esc, esh, qsc, qsh):
    full = lambda shape: pl.BlockSpec(shape, lambda i: (0,) * len(shape))
    return pl.pallas_call(
        _fin_body,
        grid=(N // BN,),
        in_specs=[
            pl.BlockSpec((BN, 1), lambda i: (i, 0)),
            pl.BlockSpec((BN, 2), lambda i: (i, 0)),
            pl.BlockSpec((BN, 2), lambda i: (i, 0)),
            pl.BlockSpec((BN, 2), lambda i: (i, 0)),
            full((FDIM, 1)), full((FDIM, 1)), full((FDIM, 1)), full((FDIM, 1)),
        ],
        out_specs=[
            pl.BlockSpec((BN, 1), lambda i: (i, 0)),
            pl.BlockSpec((BN, 1), lambda i: (i, 0)),
            pl.BlockSpec((1, 1), lambda i: (0, 0)),
        ],
        out_shape=[
            jax.ShapeDtypeStruct((N, 1), jnp.float32),
            jax.ShapeDtypeStruct((N, 1), jnp.float32),
            jax.ShapeDtypeStruct((1, 1), jnp.float32),
        ],
    )(z2, o0, o1, o2, esc, esh, qsc, qsh)


def kernel(Z, R, idx_i, idx_j, params):
    p = params
    idx_i = idx_i.astype(jnp.int32)
    idx_j = idx_j.astype(jnp.int32)
    z2 = Z.astype(jnp.int32).reshape(N, 1)
    rx, ry, rz = R[:, 0], R[:, 1], R[:, 2]

    d2 = _make_d2()(rx, ry, rz, idx_i, idx_j)
    d2c = d2.reshape(E, 1)

    centers_r = p['centers'].reshape(1, K)
    widths_r = p['widths'].reshape(1, K)
    emb_pad = jnp.zeros((FDIM, FDIM), jnp.float32).at[:95].set(p['emb'])
    pad1 = lambda t: jnp.zeros((FDIM, 1), jnp.float32).at[:95, 0].set(t)

    seg = _make_seg()

    x, xi, hj = None, None, None
    x, xi, hj = (lambda t: (t[0], t[1], t[2]))(
        _run_c0(z2, emb_pad, p['Wi'][0], p['bi'][0].reshape(1, FDIM),
                p['Wj'][0], p['bj'][0].reshape(1, FDIM)))

    outs = []
    dij_c = None
    for b in range(NB):
        g, dij_c = _run_g(d2c, centers_r, widths_r, p['k2f'][b])
        g3 = g.reshape(NCH, CE, FDIM)
        macc = seg(g3, hj, idx_i, idx_j)
        has_next = b < NB - 1
        nb = b + 1 if has_next else 0
        wts = [
            p['riW1'][b], p['rib1'][b].reshape(NRI, 1, FDIM),
            p['riW2'][b], p['rib2'][b].reshape(NRI, 1, FDIM),
            p['projW'][b], p['projb'][b].reshape(1, FDIM),
            p['u'][b].reshape(1, FDIM),
            p['raW1'][b], p['rab1'][b].reshape(NRA, 1, FDIM),
            p['raW2'][b], p['rab2'][b].reshape(NRA, 1, FDIM),
            p['roW1'][b], p['rob1'][b].reshape(NRO, 1, FDIM),
            p['roW2'][b], p['rob2'][b].reshape(NRO, 1, FDIM),
            p['outW'][b], p['outb'][b].reshape(1, 2),
            p['Wi'][nb], p['bi'][nb].reshape(1, FDIM),
            p['Wj'][nb], p['bj'][nb].reshape(1, FDIM),
        ]
        res = _run_node(has_next, xi, macc, x, wts)
        if has_next:
            out_b, x, xi, hj = res
        else:
            (out_b,) = res
        outs.append(out_b)

    ea, qa, nh = _run_fin(z2, outs[0], outs[1], outs[2],
                          pad1(p['Escale']), pad1(p['Eshift']),
                          pad1(p['Qscale']), pad1(p['Qshift']))
    return (ea.reshape(N), qa.reshape(N), dij_c.reshape(E), nh.reshape(()))
